# baseline (speedup 1.0000x reference)
import jax
import jax.numpy as jnp
from jax import lax
from jax.experimental import pallas as pl
from jax.experimental.pallas import tpu as pltpu
from jax.experimental.pallas import tpu_sc as plsc

K = 8192
B = 64
HW = 32
NPIX = HW * HW
L = 16
NC = 2
NS = 16
IMGS_PER_W = B // (NC * NS)


def _table_body(cb_ref, w_ref, b_ref, out_ref):
    logits = jnp.dot(cb_ref[...], w_ref[...],
                     preferred_element_type=jnp.float32)
    out_ref[...] = jax.nn.sigmoid(logits + b_ref[...])


def _gather_body(tbl_hbm, z_hbm,
                 endo_hbm, nuc_hbm, alea_hbm, epis_hbm,
                 tbl_v, zimg_v, oute_v, outn_v, mrow_v):
    wid = lax.axis_index("s") * NC + lax.axis_index("c")
    pltpu.sync_copy(tbl_hbm, tbl_v)
    for t in range(IMGS_PER_W):
        img = wid * IMGS_PER_W + t
        pltpu.sync_copy(z_hbm.at[pl.ds(img * HW, HW)], zimg_v)

        def body(r, accs):
            acc_e, acc_n = accs
            for c in range(HW // L):
                idx = zimg_v[r, pl.ds(c * L, L)]
                idx2 = idx + idx
                e = plsc.load_gather(tbl_v, [idx2])
                n = plsc.load_gather(tbl_v, [idx2 + 1])
                oute_v[r, pl.ds(c * L, L)] = e
                outn_v[r, pl.ds(c * L, L)] = n
                acc_e = acc_e + e
                acc_n = acc_n + n
            return (acc_e, acc_n)

        zero = jnp.zeros((L,), jnp.float32)
        acc_e, acc_n = lax.fori_loop(0, HW, body, (zero, zero))
        pltpu.sync_copy(oute_v, endo_hbm.at[img, 0])
        pltpu.sync_copy(outn_v, nuc_hbm.at[img, 0])
        mrow_v[...] = jnp.full((L,), jnp.sum(acc_e) * (1.0 / NPIX),
                               jnp.float32)
        pltpu.sync_copy(mrow_v, alea_hbm.at[img])
        mrow_v[...] = jnp.full((L,), jnp.sum(acc_n) * (1.0 / NPIX),
                               jnp.float32)
        pltpu.sync_copy(mrow_v, epis_hbm.at[img])


def kernel(z, codebook, lin_w, lin_b):
    table = pl.pallas_call(
        _table_body,
        out_shape=jax.ShapeDtypeStruct((K, 2), jnp.float32),
    )(codebook, lin_w, lin_b.reshape(1, 2))
    tbl = table.reshape(-1)
    z2 = z.reshape(B * HW, HW).astype(jnp.int32)

    mesh = plsc.VectorSubcoreMesh(core_axis_name="c", subcore_axis_name="s")
    sc = pl.kernel(
        _gather_body,
        mesh=mesh,
        compiler_params=pltpu.CompilerParams(needs_layout_passes=False),
        out_type=[
            jax.ShapeDtypeStruct((B, 1, HW, HW), jnp.float32),
            jax.ShapeDtypeStruct((B, 1, HW, HW), jnp.float32),
            jax.ShapeDtypeStruct((B, L), jnp.float32),
            jax.ShapeDtypeStruct((B, L), jnp.float32),
        ],
        scratch_types=[
            pltpu.VMEM((2 * K,), jnp.float32),
            pltpu.VMEM((HW, HW), jnp.int32),
            pltpu.VMEM((HW, HW), jnp.float32),
            pltpu.VMEM((HW, HW), jnp.float32),
            pltpu.VMEM((L,), jnp.float32),
        ],
    )
    endosome, nuclear, alea_b, epis_b = sc(tbl, z2)
    alea = alea_b[:, :1]
    epis = epis_b[:, :1]
    return (endosome, nuclear, alea, epis)


# speedup vs baseline: 1.6159x; 1.6159x over previous
import jax
import jax.numpy as jnp
from jax import lax
from jax.experimental import pallas as pl
from jax.experimental.pallas import tpu as pltpu
from jax.experimental.pallas import tpu_sc as plsc

K = 8192
B = 64
HW = 32
NPIX = HW * HW
L = 16
NC = 2
NS = 16
NG = B // L


def _table_body(cbt_ref, wt_ref, b_ref, out_ref):
    logits = lax.dot_general(wt_ref[...], cbt_ref[...],
                             (((1,), (0,)), ((), ())),
                             preferred_element_type=jnp.float32)
    out_ref[...] = jax.nn.sigmoid(logits + b_ref[...])


def _gather_body(tbl_hbm, zt_hbm,
                 endo_hbm, nuc_hbm, means_hbm,
                 tbl_e_v, tbl_n_v, zrow_v, oute_v, outn_v,
                 partial_v, red_v, mean_v, shared):
    core = lax.axis_index("c")
    sid = lax.axis_index("s")
    h = sid * NC + core
    pltpu.sync_copy(tbl_hbm.at[0], tbl_e_v)
    pltpu.sync_copy(tbl_hbm.at[1], tbl_n_v)
    pltpu.sync_copy(zt_hbm.at[h], zrow_v)

    def body(w, accs):
        accs = list(accs)
        for g in range(NG):
            idx = zrow_v[w, pl.ds(g * L, L)]
            e = plsc.load_gather(tbl_e_v, [idx])
            n = plsc.load_gather(tbl_n_v, [idx])
            oute_v[w, pl.ds(g * L, L)] = e
            outn_v[w, pl.ds(g * L, L)] = n
            accs[g] = accs[g] + e
            accs[NG + g] = accs[NG + g] + n
        return tuple(accs)

    zero = jnp.zeros((L,), jnp.float32)
    accs = lax.fori_loop(0, HW, body, (zero,) * (2 * NG))
    pltpu.sync_copy(oute_v, endo_hbm.at[h])
    pltpu.sync_copy(outn_v, nuc_hbm.at[h])
    for g in range(2 * NG):
        partial_v[pl.ds(g * L, L)] = accs[g] * (1.0 / NPIX)
    pltpu.sync_copy(partial_v, shared.at[sid])
    plsc.subcore_barrier()

    @pl.when(sid == 0)
    def _():
        pltpu.sync_copy(shared, red_v)
        for g in range(2 * NG):
            acc = red_v[0, pl.ds(g * L, L)]
            for r in range(1, NS):
                acc = acc + red_v[r, pl.ds(g * L, L)]
            mean_v[pl.ds(g * L, L)] = acc
        pltpu.sync_copy(mean_v, means_hbm.at[core])


def kernel(z, codebook, lin_w, lin_b):
    tbl = pl.pallas_call(
        _table_body,
        out_shape=jax.ShapeDtypeStruct((2, K), jnp.float32),
    )(codebook.T, lin_w.T, lin_b.reshape(2, 1))
    zt = z.transpose(1, 2, 0).astype(jnp.int32)

    mesh = plsc.VectorSubcoreMesh(core_axis_name="c", subcore_axis_name="s")
    sc = pl.kernel(
        _gather_body,
        mesh=mesh,
        compiler_params=pltpu.CompilerParams(needs_layout_passes=False),
        out_type=[
            jax.ShapeDtypeStruct((HW, HW, B), jnp.float32),
            jax.ShapeDtypeStruct((HW, HW, B), jnp.float32),
            jax.ShapeDtypeStruct((NC, 2 * B), jnp.float32),
        ],
        scratch_types=[
            pltpu.VMEM((K,), jnp.float32),
            pltpu.VMEM((K,), jnp.float32),
            pltpu.VMEM((HW, B), jnp.int32),
            pltpu.VMEM((HW, B), jnp.float32),
            pltpu.VMEM((HW, B), jnp.float32),
            pltpu.VMEM((2 * B,), jnp.float32),
            pltpu.VMEM((NS, 2 * B), jnp.float32),
            pltpu.VMEM((2 * B,), jnp.float32),
            pltpu.VMEM_SHARED((NS, 2 * B), jnp.float32),
        ],
    )
    oute, outn, means = sc(tbl, zt)
    endosome = oute.transpose(2, 0, 1).reshape(B, 1, HW, HW)
    nuclear = outn.transpose(2, 0, 1).reshape(B, 1, HW, HW)
    m = means[0] + means[1]
    alea = m[:B].reshape(B, 1)
    epis = m[B:].reshape(B, 1)
    return (endosome, nuclear, alea, epis)


# speedup vs baseline: 1.8113x; 1.1209x over previous
import jax
import jax.numpy as jnp
from jax import lax
from jax.experimental import pallas as pl
from jax.experimental.pallas import tpu as pltpu
from jax.experimental.pallas import tpu_sc as plsc

K = 8192
B = 64
HW = 32
NPIX = HW * HW
L = 16
NC = 2
NS = 16
NG = B // L


def _table_body(cbt_ref, wt_ref, b_ref, out_ref):
    logits = lax.dot_general(wt_ref[...], cbt_ref[...],
                             (((1,), (0,)), ((), ())),
                             preferred_element_type=jnp.float32)
    out_ref[0:1, :] = jax.nn.sigmoid(logits[0:1, :] + b_ref[0])
    out_ref[1:2, :] = jax.nn.sigmoid(logits[1:2, :] + b_ref[1])


def _gather_body(tbl_hbm, zt_hbm,
                 endo_hbm, nuc_hbm, means_hbm,
                 tbl_e_v, tbl_n_v, zrow_v, oute_v, outn_v,
                 partial_v, red_v, mean_v, shared, sem1, sem2, sem3):
    core = lax.axis_index("c")
    sid = lax.axis_index("s")
    h = sid * NC + core
    c1 = pltpu.async_copy(tbl_hbm.at[0], tbl_e_v, sem1)
    c2 = pltpu.async_copy(tbl_hbm.at[1], tbl_n_v, sem2)
    c3 = pltpu.async_copy(zt_hbm.at[h], zrow_v, sem3)
    c1.wait()
    c2.wait()
    c3.wait()

    def body(w, accs):
        accs = list(accs)
        for g in range(NG):
            idx = zrow_v[w, pl.ds(g * L, L)]
            e = plsc.load_gather(tbl_e_v, [idx])
            n = plsc.load_gather(tbl_n_v, [idx])
            oute_v[w, pl.ds(g * L, L)] = e
            outn_v[w, pl.ds(g * L, L)] = n
            accs[g] = accs[g] + e
            accs[NG + g] = accs[NG + g] + n
        return tuple(accs)

    zero = jnp.zeros((L,), jnp.float32)
    accs = lax.fori_loop(0, HW, body, (zero,) * (2 * NG))
    o1 = pltpu.async_copy(oute_v, endo_hbm.at[h], sem1)
    o2 = pltpu.async_copy(outn_v, nuc_hbm.at[h], sem2)
    for g in range(2 * NG):
        partial_v[pl.ds(g * L, L)] = accs[g] * (1.0 / NPIX)
    pltpu.sync_copy(partial_v, shared.at[sid])
    plsc.subcore_barrier()

    @pl.when(sid == 0)
    def _():
        pltpu.sync_copy(shared, red_v)
        for g in range(2 * NG):
            acc = red_v[0, pl.ds(g * L, L)]
            for r in range(1, NS):
                acc = acc + red_v[r, pl.ds(g * L, L)]
            mean_v[pl.ds(g * L, L)] = acc
        pltpu.sync_copy(mean_v, means_hbm.at[core])

    o1.wait()
    o2.wait()


def kernel(z, codebook, lin_w, lin_b):
    tbl = pl.pallas_call(
        _table_body,
        out_shape=jax.ShapeDtypeStruct((2, K), jnp.float32),
        in_specs=[
            pl.BlockSpec(memory_space=pltpu.VMEM),
            pl.BlockSpec(memory_space=pltpu.VMEM),
            pl.BlockSpec(memory_space=pltpu.SMEM),
        ],
    )(codebook.T, lin_w.T, lin_b)
    zt = z.transpose(1, 2, 0).astype(jnp.int32)

    mesh = plsc.VectorSubcoreMesh(core_axis_name="c", subcore_axis_name="s")
    sc = pl.kernel(
        _gather_body,
        mesh=mesh,
        compiler_params=pltpu.CompilerParams(needs_layout_passes=False),
        out_type=[
            jax.ShapeDtypeStruct((HW, HW, B), jnp.float32),
            jax.ShapeDtypeStruct((HW, HW, B), jnp.float32),
            jax.ShapeDtypeStruct((NC, 2 * B), jnp.float32),
        ],
        scratch_types=[
            pltpu.VMEM((K,), jnp.float32),
            pltpu.VMEM((K,), jnp.float32),
            pltpu.VMEM((HW, B), jnp.int32),
            pltpu.VMEM((HW, B), jnp.float32),
            pltpu.VMEM((HW, B), jnp.float32),
            pltpu.VMEM((2 * B,), jnp.float32),
            pltpu.VMEM((NS, 2 * B), jnp.float32),
            pltpu.VMEM((2 * B,), jnp.float32),
            pltpu.VMEM_SHARED((NS, 2 * B), jnp.float32),
            pltpu.SemaphoreType.DMA,
            pltpu.SemaphoreType.DMA,
            pltpu.SemaphoreType.DMA,
        ],
    )
    oute, outn, means = sc(tbl, zt)
    endosome = oute.transpose(2, 0, 1).reshape(B, 1, HW, HW)
    nuclear = outn.transpose(2, 0, 1).reshape(B, 1, HW, HW)
    alea = (means[0, :B] + means[1, :B]).reshape(B, 1)
    epis = (means[0, B:] + means[1, B:]).reshape(B, 1)
    return (endosome, nuclear, alea, epis)
